# Initial kernel scaffold; baseline (speedup 1.0000x reference)
import jax
import jax.numpy as jnp
from jax.experimental import pallas as pl


def kernel(x, edge_index, W1, b1, W2, b2):
    raise NotImplementedError("write your pallas kernel here")


# speedup vs baseline: 8.6947x; 8.6947x over previous
import functools

import jax
import jax.numpy as jnp
from jax import lax
from jax.experimental import pallas as pl
from jax.experimental.pallas import tpu as pltpu
from jax.experimental.pallas import tpu_sc as plsc

N = 10000
E = 320000
D = 128

NC = 2
NS = 16
NW = NC * NS
C = 128
K = 80
G = 16
EP = NW * K * C
NP = 10240
RPT = NP // NS

BT = 1024

@functools.lru_cache(maxsize=None)
def _sc_kernels():
    mesh = plsc.VectorSubcoreMesh(core_axis_name="c", subcore_axis_name="s",
                                  num_cores=NC, num_subcores=NS)

    @functools.partial(
        pl.kernel,
        out_type=jax.ShapeDtypeStruct((NC * NP,), jnp.float32),
        mesh=mesh,
        scratch_types=[
            pltpu.VMEM((K, C), jnp.int32),
            pltpu.VMEM((C,), jnp.float32),
            pltpu.VMEM_SHARED((NP,), jnp.float32),
            pltpu.SemaphoreType.DMA,
        ],
    )
    def deg_kernel(dst_hbm, ones_hbm, zeros_hbm, out_hbm,
                   dst_v, ones_v, acc_sh, sem):
        c = lax.axis_index("c")
        s = lax.axis_index("s")
        wid = c * NS + s
        pltpu.sync_copy(zeros_hbm, acc_sh.at[pl.ds(s * RPT, RPT)])
        pltpu.sync_copy(ones_hbm, ones_v)
        pltpu.sync_copy(dst_hbm.at[pl.ds(wid * K, K)], dst_v)
        plsc.subcore_barrier()

        def body(j, carry):
            pltpu.sync_copy(ones_v, acc_sh.at[dst_v.at[j]], add=True)
            return carry

        lax.fori_loop(0, K, body, 0)
        plsc.subcore_barrier()
        pltpu.sync_copy(acc_sh.at[pl.ds(s * RPT, RPT)],
                        out_hbm.at[pl.ds(c * NP + s * RPT, RPT)])

    @functools.partial(
        pl.kernel,
        out_type=jax.ShapeDtypeStruct((NC * NP, D), jnp.float32),
        mesh=mesh,
        scratch_types=[
            pltpu.VMEM((G, C), jnp.int32),
            pltpu.VMEM((G, C), jnp.int32),
            pltpu.VMEM((2, C, D), jnp.float32),
            pltpu.VMEM_SHARED((NP, D), jnp.float32),
            pltpu.SemaphoreType.DMA,
            pltpu.SemaphoreType.DMA,
        ],
    )
    def edge_kernel(hs_hbm, src_hbm, dst_hbm, zeros_hbm, out_hbm,
                    src_v, dst_v, rows_v, acc_sh, g0, g1):
        c = lax.axis_index("c")
        s = lax.axis_index("s")
        wid = c * NS + s
        pltpu.sync_copy(zeros_hbm, acc_sh.at[pl.ds(s * RPT, RPT)])
        plsc.subcore_barrier()

        def group(g, carry):
            pltpu.sync_copy(src_hbm.at[pl.ds(wid * K + g * G, G)], src_v)
            pltpu.sync_copy(dst_hbm.at[pl.ds(wid * K + g * G, G)], dst_v)

            def body(jj, carry2):
                j = 2 * jj
                d0 = pltpu.async_copy(hs_hbm.at[src_v.at[j]], rows_v.at[0], g0)
                d1 = pltpu.async_copy(hs_hbm.at[src_v.at[j + 1]], rows_v.at[1],
                                      g1)
                d0.wait()
                pltpu.sync_copy(rows_v.at[0], acc_sh.at[dst_v.at[j]], add=True)
                d1.wait()
                pltpu.sync_copy(rows_v.at[1], acc_sh.at[dst_v.at[j + 1]],
                                add=True)
                return carry2

            lax.fori_loop(0, G // 2, body, 0)
            return carry

        lax.fori_loop(0, K // G, group, 0)
        plsc.subcore_barrier()
        pltpu.sync_copy(acc_sh.at[pl.ds(s * RPT, RPT)],
                        out_hbm.at[pl.ds(c * NP + s * RPT, RPT)])

    return deg_kernel, edge_kernel


def _tc1_body(x_ref, w_ref, d0_ref, d1_ref, hs_ref, dinv_ref):
    dinv = lax.rsqrt(d0_ref[...] + d1_ref[...] + 1.0)
    h = jnp.dot(x_ref[...], w_ref[...], preferred_element_type=jnp.float32)
    hs_ref[...] = h * dinv
    dinv_ref[...] = dinv


_tc1 = pl.pallas_call(
    _tc1_body,
    grid=(NP // BT,),
    in_specs=[
        pl.BlockSpec((BT, D), lambda i: (i, 0)),
        pl.BlockSpec((D, D), lambda i: (0, 0)),
        pl.BlockSpec((BT, 1), lambda i: (i, 0)),
        pl.BlockSpec((BT, 1), lambda i: (i, 0)),
    ],
    out_specs=[
        pl.BlockSpec((BT, D), lambda i: (i, 0)),
        pl.BlockSpec((BT, 1), lambda i: (i, 0)),
    ],
    out_shape=[
        jax.ShapeDtypeStruct((NP, D), jnp.float32),
        jax.ShapeDtypeStruct((NP, 1), jnp.float32),
    ],
)


def _tc2_body(a0_ref, a1_ref, hs1_ref, dinv_ref, b_ref, w_ref, hs2_ref):
    t = (a0_ref[...] + a1_ref[...] + hs1_ref[...]) * dinv_ref[...] + b_ref[...]
    t = jnp.maximum(t, 0.0)
    hs2_ref[...] = jnp.dot(t, w_ref[...],
                           preferred_element_type=jnp.float32) * dinv_ref[...]


_tc2 = pl.pallas_call(
    _tc2_body,
    grid=(NP // BT,),
    in_specs=[
        pl.BlockSpec((BT, D), lambda i: (i, 0)),
        pl.BlockSpec((BT, D), lambda i: (i, 0)),
        pl.BlockSpec((BT, D), lambda i: (i, 0)),
        pl.BlockSpec((BT, 1), lambda i: (i, 0)),
        pl.BlockSpec((1, D), lambda i: (0, 0)),
        pl.BlockSpec((D, D), lambda i: (0, 0)),
    ],
    out_specs=pl.BlockSpec((BT, D), lambda i: (i, 0)),
    out_shape=jax.ShapeDtypeStruct((NP, D), jnp.float32),
)


def _tc3_body(a0_ref, a1_ref, hs2_ref, dinv_ref, b_ref, out_ref):
    out_ref[...] = ((a0_ref[...] + a1_ref[...] + hs2_ref[...])
                    * dinv_ref[...] + b_ref[...])


_tc3 = pl.pallas_call(
    _tc3_body,
    grid=(NP // BT,),
    in_specs=[
        pl.BlockSpec((BT, D), lambda i: (i, 0)),
        pl.BlockSpec((BT, D), lambda i: (i, 0)),
        pl.BlockSpec((BT, D), lambda i: (i, 0)),
        pl.BlockSpec((BT, 1), lambda i: (i, 0)),
        pl.BlockSpec((1, D), lambda i: (0, 0)),
    ],
    out_specs=pl.BlockSpec((BT, D), lambda i: (i, 0)),
    out_shape=jax.ShapeDtypeStruct((NP, D), jnp.float32),
)


def kernel(x, edge_index, W1, b1, W2, b2):
    xp = jnp.pad(x, ((0, NP - N), (0, 0)))
    pad = jnp.full((EP - E,), NP - 1, dtype=jnp.int32)
    src2 = jnp.concatenate([edge_index[0], pad]).reshape(NW * K, C)
    dst2 = jnp.concatenate([edge_index[1], pad]).reshape(NW * K, C)
    zrows = jnp.zeros((RPT, D), jnp.float32)
    zvec = jnp.zeros((RPT,), jnp.float32)
    ones = jnp.ones((C,), jnp.float32)

    _deg_kernel, _edge_kernel = _sc_kernels()
    deg = _deg_kernel(dst2, ones, zvec)
    d0 = deg[:NP].reshape(NP, 1)
    d1 = deg[NP:].reshape(NP, 1)

    hs1, dinv = _tc1(xp, W1, d0, d1)
    agg1 = _edge_kernel(hs1, src2, dst2, zrows)
    hs2 = _tc2(agg1[:NP], agg1[NP:], hs1, dinv, b1.reshape(1, D), W2)
    agg2 = _edge_kernel(hs2, src2, dst2, zrows)
    out = _tc3(agg2[:NP], agg2[NP:], hs2, dinv, b2.reshape(1, D))
    return out[:N]


# speedup vs baseline: 23.2479x; 2.6738x over previous
import functools

import jax
import jax.numpy as jnp
from jax import lax
from jax.experimental import pallas as pl
from jax.experimental.pallas import tpu as pltpu
from jax.experimental.pallas import tpu_sc as plsc

N = 10000
E = 320000
D = 128

NC = 2
NS = 16
NW = NC * NS
C = 128
K = 80
G = 16
EP = NW * K * C
NP = 10240
RPT = NP // NS

BT = 1024

@functools.lru_cache(maxsize=None)
def _sc_kernels():
    mesh = plsc.VectorSubcoreMesh(core_axis_name="c", subcore_axis_name="s",
                                  num_cores=NC, num_subcores=NS)

    @functools.partial(
        pl.kernel,
        out_type=jax.ShapeDtypeStruct((NC * NP,), jnp.float32),
        mesh=mesh,
        scratch_types=[
            pltpu.VMEM((K, C), jnp.int32),
            pltpu.VMEM((C,), jnp.float32),
            pltpu.VMEM_SHARED((NP,), jnp.float32),
            pltpu.SemaphoreType.DMA,
        ],
    )
    def deg_kernel(dst_hbm, ones_hbm, zeros_hbm, out_hbm,
                   dst_v, ones_v, acc_sh, sem):
        c = lax.axis_index("c")
        s = lax.axis_index("s")
        wid = c * NS + s
        pltpu.sync_copy(zeros_hbm, acc_sh.at[pl.ds(s * RPT, RPT)])
        pltpu.sync_copy(ones_hbm, ones_v)
        pltpu.sync_copy(dst_hbm.at[pl.ds(wid * K, K)], dst_v)
        plsc.subcore_barrier()

        def body(j, carry):
            pltpu.sync_copy(ones_v, acc_sh.at[dst_v.at[j]], add=True)
            return carry

        lax.fori_loop(0, K, body, 0)
        plsc.subcore_barrier()
        pltpu.sync_copy(acc_sh.at[pl.ds(s * RPT, RPT)],
                        out_hbm.at[pl.ds(c * NP + s * RPT, RPT)])

    @functools.partial(
        pl.kernel,
        out_type=jax.ShapeDtypeStruct((NC * NP, D), jnp.float32),
        mesh=mesh,
        scratch_types=[
            pltpu.VMEM((G, C), jnp.int32),
            pltpu.VMEM((G, C), jnp.int32),
            pltpu.VMEM((2, C, D), jnp.float32),
            pltpu.VMEM_SHARED((NP, D), jnp.float32),
            pltpu.SemaphoreType.DMA,
            pltpu.SemaphoreType.DMA,
        ],
    )
    def edge_kernel(hs_hbm, src_hbm, dst_hbm, zeros_hbm, out_hbm,
                    src_v, dst_v, rows_v, acc_sh, g0, g1):
        c = lax.axis_index("c")
        s = lax.axis_index("s")
        wid = c * NS + s
        pltpu.sync_copy(zeros_hbm, acc_sh.at[pl.ds(s * RPT, RPT)])
        plsc.subcore_barrier()

        def group(g, carry):
            pltpu.sync_copy(src_hbm.at[pl.ds(wid * K + g * G, G)], src_v)
            pltpu.sync_copy(dst_hbm.at[pl.ds(wid * K + g * G, G)], dst_v)

            def body(jj, carry2):
                j = 2 * jj
                d0 = pltpu.async_copy(hs_hbm.at[src_v.at[j]], rows_v.at[0], g0)
                d1 = pltpu.async_copy(hs_hbm.at[src_v.at[j + 1]], rows_v.at[1],
                                      g1)
                d0.wait()
                pltpu.sync_copy(rows_v.at[0], acc_sh.at[dst_v.at[j]], add=True)
                d1.wait()
                pltpu.sync_copy(rows_v.at[1], acc_sh.at[dst_v.at[j + 1]],
                                add=True)
                return carry2

            lax.fori_loop(0, G // 2, body, 0)
            return carry

        lax.fori_loop(0, K // G, group, 0)
        plsc.subcore_barrier()
        pltpu.sync_copy(acc_sh.at[pl.ds(s * RPT, RPT)],
                        out_hbm.at[pl.ds(c * NP + s * RPT, RPT)])

    return deg_kernel, edge_kernel


def _tc1_body(x_ref, w_ref, d0_ref, d1_ref, hs_ref, dinv_ref):
    dinv = lax.rsqrt(d0_ref[...] + d1_ref[...] + 1.0)
    h = jnp.dot(x_ref[...], w_ref[...], preferred_element_type=jnp.float32)
    hs_ref[...] = h * dinv
    dinv_ref[...] = dinv


_tc1 = pl.pallas_call(
    _tc1_body,
    grid=(NP // BT,),
    in_specs=[
        pl.BlockSpec((BT, D), lambda i: (i, 0)),
        pl.BlockSpec((D, D), lambda i: (0, 0)),
        pl.BlockSpec((BT, 1), lambda i: (i, 0)),
        pl.BlockSpec((BT, 1), lambda i: (i, 0)),
    ],
    out_specs=[
        pl.BlockSpec((BT, D), lambda i: (i, 0)),
        pl.BlockSpec((BT, 1), lambda i: (i, 0)),
    ],
    out_shape=[
        jax.ShapeDtypeStruct((NP, D), jnp.float32),
        jax.ShapeDtypeStruct((NP, 1), jnp.float32),
    ],
)


def _tc2_body(a0_ref, a1_ref, hs1_ref, dinv_ref, b_ref, w_ref, hs2_ref):
    t = (a0_ref[...] + a1_ref[...] + hs1_ref[...]) * dinv_ref[...] + b_ref[...]
    t = jnp.maximum(t, 0.0)
    hs2_ref[...] = jnp.dot(t, w_ref[...],
                           preferred_element_type=jnp.float32) * dinv_ref[...]


_tc2 = pl.pallas_call(
    _tc2_body,
    grid=(NP // BT,),
    in_specs=[
        pl.BlockSpec((BT, D), lambda i: (i, 0)),
        pl.BlockSpec((BT, D), lambda i: (i, 0)),
        pl.BlockSpec((BT, D), lambda i: (i, 0)),
        pl.BlockSpec((BT, 1), lambda i: (i, 0)),
        pl.BlockSpec((1, D), lambda i: (0, 0)),
        pl.BlockSpec((D, D), lambda i: (0, 0)),
    ],
    out_specs=pl.BlockSpec((BT, D), lambda i: (i, 0)),
    out_shape=jax.ShapeDtypeStruct((NP, D), jnp.float32),
)


def _tc3_body(a0_ref, a1_ref, hs2_ref, dinv_ref, b_ref, out_ref):
    out_ref[...] = ((a0_ref[...] + a1_ref[...] + hs2_ref[...])
                    * dinv_ref[...] + b_ref[...])


_tc3 = pl.pallas_call(
    _tc3_body,
    grid=(NP // BT,),
    in_specs=[
        pl.BlockSpec((BT, D), lambda i: (i, 0)),
        pl.BlockSpec((BT, D), lambda i: (i, 0)),
        pl.BlockSpec((BT, D), lambda i: (i, 0)),
        pl.BlockSpec((BT, 1), lambda i: (i, 0)),
        pl.BlockSpec((1, D), lambda i: (0, 0)),
    ],
    out_specs=pl.BlockSpec((BT, D), lambda i: (i, 0)),
    out_shape=jax.ShapeDtypeStruct((NP, D), jnp.float32),
)


def kernel(x, edge_index, W1, b1, W2, b2):
    xp = jnp.pad(x, ((0, NP - N), (0, 0)))
    pad = N + (jnp.arange(EP - E, dtype=jnp.int32) % (NP - N))
    src2 = jnp.concatenate([edge_index[0], pad]).reshape(NW * K, C)
    dst2 = jnp.concatenate([edge_index[1], pad]).reshape(NW * K, C)
    zrows = jnp.zeros((RPT, D), jnp.float32)
    zvec = jnp.zeros((RPT,), jnp.float32)
    ones = jnp.ones((C,), jnp.float32)

    _deg_kernel, _edge_kernel = _sc_kernels()
    deg = _deg_kernel(dst2, ones, zvec)
    d0 = deg[:NP].reshape(NP, 1)
    d1 = deg[NP:].reshape(NP, 1)

    hs1, dinv = _tc1(xp, W1, d0, d1)
    agg1 = _edge_kernel(hs1, src2, dst2, zrows)
    hs2 = _tc2(agg1[:NP], agg1[NP:], hs1, dinv, b1.reshape(1, D), W2)
    agg2 = _edge_kernel(hs2, src2, dst2, zrows)
    out = _tc3(agg2[:NP], agg2[NP:], hs2, dinv, b2.reshape(1, D))
    return out[:N]


# speedup vs baseline: 28.6155x; 1.2309x over previous
import functools

import jax
import jax.numpy as jnp
from jax import lax
from jax.experimental import pallas as pl
from jax.experimental.pallas import tpu as pltpu
from jax.experimental.pallas import tpu_sc as plsc

N = 10000
E = 320000
D = 128

NC = 2
NS = 16
NW = NC * NS
C = 64
K = 160
G = 16
EP = NW * K * C
NP = 10240
RPT = NP // NS

BT = 1024

NSLOT = 4


@functools.lru_cache(maxsize=None)
def _sc_kernels():
    mesh = plsc.VectorSubcoreMesh(core_axis_name="c", subcore_axis_name="s",
                                  num_cores=NC, num_subcores=NS)

    @functools.partial(
        pl.kernel,
        out_type=jax.ShapeDtypeStruct((NC * NP,), jnp.float32),
        mesh=mesh,
        scratch_types=[
            pltpu.VMEM((K, C), jnp.int32),
            pltpu.VMEM((C,), jnp.float32),
            pltpu.VMEM_SHARED((NP,), jnp.float32),
            pltpu.SemaphoreType.DMA,
        ],
    )
    def deg_kernel(dst_hbm, ones_hbm, zeros_hbm, out_hbm,
                   dst_v, ones_v, acc_sh, sem):
        c = lax.axis_index("c")
        s = lax.axis_index("s")
        wid = c * NS + s
        pltpu.sync_copy(zeros_hbm, acc_sh.at[pl.ds(s * RPT, RPT)])
        pltpu.sync_copy(ones_hbm, ones_v)
        pltpu.sync_copy(dst_hbm.at[wid], dst_v)
        plsc.subcore_barrier()

        def body(j, carry):
            pltpu.sync_copy(ones_v, acc_sh.at[dst_v.at[j]], add=True)
            return carry

        lax.fori_loop(0, K, body, 0)
        plsc.subcore_barrier()
        pltpu.sync_copy(acc_sh.at[pl.ds(s * RPT, RPT)],
                        out_hbm.at[pl.ds(c * NP + s * RPT, RPT)])

    @functools.partial(
        pl.kernel,
        out_type=jax.ShapeDtypeStruct((NC * NP, D), jnp.float32),
        mesh=mesh,
        scratch_types=[
            pltpu.VMEM((2, G, C), jnp.int32),
            pltpu.VMEM((2, G, C), jnp.int32),
            pltpu.VMEM((NSLOT, C, D), jnp.float32),
            pltpu.VMEM_SHARED((NP, D), jnp.float32),
            pltpu.SemaphoreType.DMA,
            pltpu.SemaphoreType.DMA,
            pltpu.SemaphoreType.DMA,
            pltpu.SemaphoreType.DMA,
            pltpu.SemaphoreType.DMA,
            pltpu.SemaphoreType.DMA,
            pltpu.SemaphoreType.DMA,
            pltpu.SemaphoreType.DMA,
        ],
    )
    def edge_kernel(hs_hbm, src_hbm, dst_hbm, zeros_hbm, out_hbm,
                    src_v, dst_v, rows_v, acc_sh,
                    g0, g1, g2, g3, s0, s1, s2, s3):
        c = lax.axis_index("c")
        s = lax.axis_index("s")
        wid = c * NS + s
        gsems = (g0, g1, g2, g3)
        ssems = (s0, s1, s2, s3)

        pltpu.sync_copy(zeros_hbm, acc_sh.at[pl.ds(s * RPT, RPT)])
        pltpu.sync_copy(src_hbm.at[wid, pl.ds(0, G)], src_v.at[0])
        pltpu.sync_copy(dst_hbm.at[wid, pl.ds(0, G)], dst_v.at[0])
        plsc.subcore_barrier()

        def src_at(j):
            return src_v.at[lax.rem(lax.div(j, G), 2), lax.rem(j, G)]

        def dst_at(j):
            return dst_v.at[lax.rem(lax.div(j, G), 2), lax.rem(j, G)]

        pltpu.async_copy(hs_hbm.at[src_at(0)], rows_v.at[0], g0)
        pltpu.async_copy(hs_hbm.at[src_at(1)], rows_v.at[1], g1)

        def body(t, carry):
            for u in range(4):
                j = 4 * t + u
                mg = (u + 2) % 4

                @pl.when(jnp.logical_and(lax.rem(j, G) == 2, j < K - G))
                def _():
                    nxt = lax.div(j, G) + 1
                    buf = lax.rem(nxt, 2)
                    pltpu.sync_copy(src_hbm.at[wid, pl.ds(nxt * G, G)],
                                    src_v.at[buf])
                    pltpu.sync_copy(dst_hbm.at[wid, pl.ds(nxt * G, G)],
                                    dst_v.at[buf])

                @pl.when(j >= 2)
                def _():
                    pltpu.make_async_copy(rows_v.at[mg],
                                          acc_sh.at[dst_at(j)],
                                          ssems[mg]).wait()

                @pl.when(j + 2 < K)
                def _():
                    pltpu.async_copy(hs_hbm.at[src_at(j + 2)],
                                     rows_v.at[mg], gsems[mg])

                pltpu.make_async_copy(hs_hbm.at[src_at(j)],
                                      rows_v.at[u], gsems[u]).wait()
                pltpu.async_copy(rows_v.at[u], acc_sh.at[dst_at(j)],
                                 ssems[u], add=True)
            return carry

        lax.fori_loop(0, K // 4, body, 0)
        pltpu.make_async_copy(rows_v.at[(K - 2) % 4],
                              acc_sh.at[dst_at(0)],
                              ssems[(K - 2) % 4]).wait()
        pltpu.make_async_copy(rows_v.at[(K - 1) % 4],
                              acc_sh.at[dst_at(0)],
                              ssems[(K - 1) % 4]).wait()
        plsc.subcore_barrier()
        pltpu.sync_copy(acc_sh.at[pl.ds(s * RPT, RPT)],
                        out_hbm.at[pl.ds(c * NP + s * RPT, RPT)])

    return deg_kernel, edge_kernel


def _tc1_body(x_ref, w_ref, d0_ref, d1_ref, hs_ref, dinv_ref):
    dinv = lax.rsqrt(d0_ref[...] + d1_ref[...] + 1.0)
    h = jnp.dot(x_ref[...], w_ref[...], preferred_element_type=jnp.float32)
    hs_ref[...] = h * dinv
    dinv_ref[...] = dinv


_tc1 = pl.pallas_call(
    _tc1_body,
    grid=(NP // BT,),
    in_specs=[
        pl.BlockSpec((BT, D), lambda i: (i, 0)),
        pl.BlockSpec((D, D), lambda i: (0, 0)),
        pl.BlockSpec((BT, 1), lambda i: (i, 0)),
        pl.BlockSpec((BT, 1), lambda i: (i, 0)),
    ],
    out_specs=[
        pl.BlockSpec((BT, D), lambda i: (i, 0)),
        pl.BlockSpec((BT, 1), lambda i: (i, 0)),
    ],
    out_shape=[
        jax.ShapeDtypeStruct((NP, D), jnp.float32),
        jax.ShapeDtypeStruct((NP, 1), jnp.float32),
    ],
)


def _tc2_body(a0_ref, a1_ref, hs1_ref, dinv_ref, b_ref, w_ref, hs2_ref):
    t = (a0_ref[...] + a1_ref[...] + hs1_ref[...]) * dinv_ref[...] + b_ref[...]
    t = jnp.maximum(t, 0.0)
    hs2_ref[...] = jnp.dot(t, w_ref[...],
                           preferred_element_type=jnp.float32) * dinv_ref[...]


_tc2 = pl.pallas_call(
    _tc2_body,
    grid=(NP // BT,),
    in_specs=[
        pl.BlockSpec((BT, D), lambda i: (i, 0)),
        pl.BlockSpec((BT, D), lambda i: (i, 0)),
        pl.BlockSpec((BT, D), lambda i: (i, 0)),
        pl.BlockSpec((BT, 1), lambda i: (i, 0)),
        pl.BlockSpec((1, D), lambda i: (0, 0)),
        pl.BlockSpec((D, D), lambda i: (0, 0)),
    ],
    out_specs=pl.BlockSpec((BT, D), lambda i: (i, 0)),
    out_shape=jax.ShapeDtypeStruct((NP, D), jnp.float32),
)


def _tc3_body(a0_ref, a1_ref, hs2_ref, dinv_ref, b_ref, out_ref):
    out_ref[...] = ((a0_ref[...] + a1_ref[...] + hs2_ref[...])
                    * dinv_ref[...] + b_ref[...])


_tc3 = pl.pallas_call(
    _tc3_body,
    grid=(NP // BT,),
    in_specs=[
        pl.BlockSpec((BT, D), lambda i: (i, 0)),
        pl.BlockSpec((BT, D), lambda i: (i, 0)),
        pl.BlockSpec((BT, D), lambda i: (i, 0)),
        pl.BlockSpec((BT, 1), lambda i: (i, 0)),
        pl.BlockSpec((1, D), lambda i: (0, 0)),
    ],
    out_specs=pl.BlockSpec((BT, D), lambda i: (i, 0)),
    out_shape=jax.ShapeDtypeStruct((NP, D), jnp.float32),
)


def kernel(x, edge_index, W1, b1, W2, b2):
    xp = jnp.pad(x, ((0, NP - N), (0, 0)))
    pad = N + (jnp.arange(EP - E, dtype=jnp.int32) % (NP - N))
    src2 = jnp.concatenate([edge_index[0], pad]).reshape(NW, K, C)
    dst2 = jnp.concatenate([edge_index[1], pad]).reshape(NW, K, C)
    zrows = jnp.zeros((RPT, D), jnp.float32)
    zvec = jnp.zeros((RPT,), jnp.float32)
    ones = jnp.ones((C,), jnp.float32)

    _deg_kernel, _edge_kernel = _sc_kernels()
    deg = _deg_kernel(dst2, ones, zvec)
    d0 = deg[:NP].reshape(NP, 1)
    d1 = deg[NP:].reshape(NP, 1)

    hs1, dinv = _tc1(xp, W1, d0, d1)
    agg1 = _edge_kernel(hs1, src2, dst2, zrows)
    hs2 = _tc2(agg1[:NP], agg1[NP:], hs1, dinv, b1.reshape(1, D), W2)
    agg2 = _edge_kernel(hs2, src2, dst2, zrows)
    out = _tc3(agg2[:NP], agg2[NP:], hs2, dinv, b2.reshape(1, D))
    return out[:N]


# speedup vs baseline: 30.4704x; 1.0648x over previous
import functools

import jax
import jax.numpy as jnp
from jax import lax
from jax.experimental import pallas as pl
from jax.experimental.pallas import tpu as pltpu
from jax.experimental.pallas import tpu_sc as plsc

N = 10000
E = 320000
D = 128

NC = 2
NS = 16
NW = NC * NS
C = 64
K = 160
G = 16
EP = NW * K * C
NP = 10240
RPT = NP // NS
NB = NP // 1024

BT = 1024

NSLOT = 4


@functools.lru_cache(maxsize=None)
def _sc_kernels():
    mesh = plsc.VectorSubcoreMesh(core_axis_name="c", subcore_axis_name="s",
                                  num_cores=NC, num_subcores=NS)

    @functools.partial(
        pl.kernel,
        out_type=jax.ShapeDtypeStruct((NC * NP,), jnp.float32),
        mesh=mesh,
        scratch_types=[
            pltpu.VMEM((K, C), jnp.int32),
            pltpu.VMEM((C,), jnp.float32),
            pltpu.VMEM_SHARED((NP,), jnp.float32),
            pltpu.SemaphoreType.DMA,
        ],
    )
    def deg_kernel(dst_hbm, ones_hbm, zeros_hbm, out_hbm,
                   dst_v, ones_v, acc_sh, sem):
        c = lax.axis_index("c")
        s = lax.axis_index("s")
        wid = c * NS + s
        pltpu.sync_copy(zeros_hbm, acc_sh.at[pl.ds(s * RPT, RPT)])
        pltpu.sync_copy(ones_hbm, ones_v)
        pltpu.sync_copy(dst_hbm.at[wid], dst_v)
        plsc.subcore_barrier()

        def group(g, carry):
            def fire(j2, carry2):
                pltpu.async_copy(ones_v, acc_sh.at[dst_v.at[g * G + j2]],
                                 sem, add=True)
                return carry2

            lax.fori_loop(0, G, fire, 0)

            @pl.when(g > 0)
            def _():
                def drain(j2, carry2):
                    pltpu.make_async_copy(ones_v, acc_sh.at[dst_v.at[0]],
                                          sem).wait()
                    return carry2

                lax.fori_loop(0, G, drain, 0)

            return carry

        lax.fori_loop(0, K // G, group, 0)

        def drain(j2, carry2):
            pltpu.make_async_copy(ones_v, acc_sh.at[dst_v.at[0]], sem).wait()
            return carry2

        lax.fori_loop(0, G, drain, 0)
        plsc.subcore_barrier()
        pltpu.sync_copy(acc_sh.at[pl.ds(s * RPT, RPT)],
                        out_hbm.at[pl.ds(c * NP + s * RPT, RPT)])

    @functools.partial(
        pl.kernel,
        out_type=jax.ShapeDtypeStruct((NC * NP, D), jnp.float32),
        mesh=mesh,
        scratch_types=[
            pltpu.VMEM((2, G, C), jnp.int32),
            pltpu.VMEM((2, G, C), jnp.int32),
            pltpu.VMEM((NSLOT, C, D), jnp.float32),
            pltpu.VMEM_SHARED((NP, D), jnp.float32),
            pltpu.SemaphoreType.DMA,
            pltpu.SemaphoreType.DMA,
            pltpu.SemaphoreType.DMA,
            pltpu.SemaphoreType.DMA,
            pltpu.SemaphoreType.DMA,
            pltpu.SemaphoreType.DMA,
            pltpu.SemaphoreType.DMA,
            pltpu.SemaphoreType.DMA,
        ],
    )
    def edge_kernel(hs_hbm, src_hbm, dst_hbm, zeros_hbm, out_hbm,
                    src_v, dst_v, rows_v, acc_sh,
                    g0, g1, g2, g3, s0, s1, s2, s3):
        c = lax.axis_index("c")
        s = lax.axis_index("s")
        wid = c * NS + s
        gsems = (g0, g1, g2, g3)
        ssems = (s0, s1, s2, s3)

        pltpu.sync_copy(zeros_hbm, acc_sh.at[pl.ds(s * RPT, RPT)])
        pltpu.sync_copy(src_hbm.at[wid, pl.ds(0, G)], src_v.at[0])
        pltpu.sync_copy(dst_hbm.at[wid, pl.ds(0, G)], dst_v.at[0])
        plsc.subcore_barrier()

        def src_at(j):
            return src_v.at[lax.rem(lax.div(j, G), 2), lax.rem(j, G)]

        def dst_at(j):
            return dst_v.at[lax.rem(lax.div(j, G), 2), lax.rem(j, G)]

        pltpu.async_copy(hs_hbm.at[src_at(0)], rows_v.at[0], g0)
        pltpu.async_copy(hs_hbm.at[src_at(1)], rows_v.at[1], g1)

        def body(t, carry):
            for u in range(4):
                j = 4 * t + u
                mg = (u + 2) % 4

                @pl.when(jnp.logical_and(lax.rem(j, G) == 2, j < K - G))
                def _():
                    nxt = lax.div(j, G) + 1
                    buf = lax.rem(nxt, 2)
                    pltpu.sync_copy(src_hbm.at[wid, pl.ds(nxt * G, G)],
                                    src_v.at[buf])
                    pltpu.sync_copy(dst_hbm.at[wid, pl.ds(nxt * G, G)],
                                    dst_v.at[buf])

                @pl.when(j >= 2)
                def _():
                    pltpu.make_async_copy(rows_v.at[mg],
                                          acc_sh.at[dst_at(j)],
                                          ssems[mg]).wait()

                @pl.when(j + 2 < K)
                def _():
                    pltpu.async_copy(hs_hbm.at[src_at(j + 2)],
                                     rows_v.at[mg], gsems[mg])

                pltpu.make_async_copy(hs_hbm.at[src_at(j)],
                                      rows_v.at[u], gsems[u]).wait()
                pltpu.async_copy(rows_v.at[u], acc_sh.at[dst_at(j)],
                                 ssems[u], add=True)
            return carry

        lax.fori_loop(0, K // 4, body, 0)
        pltpu.make_async_copy(rows_v.at[(K - 2) % 4],
                              acc_sh.at[dst_at(0)],
                              ssems[(K - 2) % 4]).wait()
        pltpu.make_async_copy(rows_v.at[(K - 1) % 4],
                              acc_sh.at[dst_at(0)],
                              ssems[(K - 1) % 4]).wait()
        plsc.subcore_barrier()
        pltpu.sync_copy(acc_sh.at[pl.ds(s * RPT, RPT)],
                        out_hbm.at[pl.ds(c * NP + s * RPT, RPT)])

    return deg_kernel, edge_kernel


def _tch_body(x_ref, w_ref, h_ref):
    h_ref[...] = jnp.dot(x_ref[...], w_ref[...],
                         preferred_element_type=jnp.float32)


_tch = pl.pallas_call(
    _tch_body,
    grid=(NB,),
    in_specs=[
        pl.BlockSpec((BT, D), lambda i: (i, 0)),
        pl.BlockSpec((D, D), lambda i: (0, 0)),
    ],
    out_specs=pl.BlockSpec((BT, D), lambda i: (i, 0)),
    out_shape=jax.ShapeDtypeStruct((NP, D), jnp.float32),
)


def _tc1_body(h_ref, d0_ref, d1_ref, hs_ref, dinv_ref):
    dinv = lax.rsqrt(d0_ref[...] + d1_ref[...] + 1.0)
    hs_ref[...] = h_ref[...] * dinv
    dinv_ref[...] = dinv


_tc1 = pl.pallas_call(
    _tc1_body,
    grid=(NB,),
    in_specs=[
        pl.BlockSpec((BT, D), lambda i: (i, 0)),
        pl.BlockSpec((BT, 1), lambda i: (i, 0)),
        pl.BlockSpec((BT, 1), lambda i: (i + NB, 0)),
    ],
    out_specs=[
        pl.BlockSpec((BT, D), lambda i: (i, 0)),
        pl.BlockSpec((BT, 1), lambda i: (i, 0)),
    ],
    out_shape=[
        jax.ShapeDtypeStruct((NP, D), jnp.float32),
        jax.ShapeDtypeStruct((NP, 1), jnp.float32),
    ],
)


def _tc2_body(a0_ref, a1_ref, hs1_ref, dinv_ref, b_ref, w_ref, hs2_ref):
    t = (a0_ref[...] + a1_ref[...] + hs1_ref[...]) * dinv_ref[...] + b_ref[...]
    t = jnp.maximum(t, 0.0)
    hs2_ref[...] = jnp.dot(t, w_ref[...],
                           preferred_element_type=jnp.float32) * dinv_ref[...]


_tc2 = pl.pallas_call(
    _tc2_body,
    grid=(NB,),
    in_specs=[
        pl.BlockSpec((BT, D), lambda i: (i, 0)),
        pl.BlockSpec((BT, D), lambda i: (i + NB, 0)),
        pl.BlockSpec((BT, D), lambda i: (i, 0)),
        pl.BlockSpec((BT, 1), lambda i: (i, 0)),
        pl.BlockSpec((1, D), lambda i: (0, 0)),
        pl.BlockSpec((D, D), lambda i: (0, 0)),
    ],
    out_specs=pl.BlockSpec((BT, D), lambda i: (i, 0)),
    out_shape=jax.ShapeDtypeStruct((NP, D), jnp.float32),
)


def _tc3_body(a0_ref, a1_ref, hs2_ref, dinv_ref, b_ref, out_ref):
    out_ref[...] = ((a0_ref[...] + a1_ref[...] + hs2_ref[...])
                    * dinv_ref[...] + b_ref[...])


_tc3 = pl.pallas_call(
    _tc3_body,
    grid=(NB,),
    in_specs=[
        pl.BlockSpec((BT, D), lambda i: (i, 0)),
        pl.BlockSpec((BT, D), lambda i: (i + NB, 0)),
        pl.BlockSpec((BT, D), lambda i: (i, 0)),
        pl.BlockSpec((BT, 1), lambda i: (i, 0)),
        pl.BlockSpec((1, D), lambda i: (0, 0)),
    ],
    out_specs=pl.BlockSpec((BT, D), lambda i: (i, 0)),
    out_shape=jax.ShapeDtypeStruct((NP, D), jnp.float32),
)


def kernel(x, edge_index, W1, b1, W2, b2):
    xp = jnp.pad(x, ((0, NP - N), (0, 0)))
    pad = N + (jnp.arange(EP - E, dtype=jnp.int32) % (NP - N))
    src2 = jnp.concatenate([edge_index[0], pad]).reshape(NW, K, C)
    dst2 = jnp.concatenate([edge_index[1], pad]).reshape(NW, K, C)
    zrows = jnp.zeros((RPT, D), jnp.float32)
    zvec = jnp.zeros((RPT,), jnp.float32)
    ones = jnp.ones((C,), jnp.float32)

    _deg_kernel, _edge_kernel = _sc_kernels()
    deg = _deg_kernel(dst2, ones, zvec)
    h1 = _tch(xp, W1)
    deg2 = deg.reshape(NC * NP, 1)

    hs1, dinv = _tc1(h1, deg2, deg2)
    agg1 = _edge_kernel(hs1, src2, dst2, zrows)
    hs2 = _tc2(agg1, agg1, hs1, dinv, b1.reshape(1, D), W2)
    agg2 = _edge_kernel(hs2, src2, dst2, zrows)
    out = _tc3(agg2, agg2, hs2, dinv, b2.reshape(1, D))
    return out[:N]
